# baseline (speedup 1.0000x reference)
import functools

import numpy as np
import jax
import jax.numpy as jnp
from jax import lax
from jax.experimental import pallas as pl
from jax.experimental.pallas import tpu as pltpu
from jax.experimental.pallas import tpu_sc as plsc

B, D, H, W = 8, 128, 224, 224
HW = H * W
T = 16
CH = 24
NPC = CH * CH
N = B * NPC
POS_R = 3

_pos = np.array([(i, j) for i in range(-3, 4) for j in range(-3, 4)
                 if i * i + j * j <= 9], dtype=np.int32).reshape(-1, 2).T
_neg = np.array([(i, j) for i in range(-8, 9, 2) for j in range(-8, 9, 2)
                 if 49 <= i * i + j * j <= 64], dtype=np.int32).reshape(-1, 2).T
P = _pos.shape[1]
NN = _neg.shape[1]
NP48 = 48
_alldx = np.zeros(NP48, np.int32)
_alldy = np.zeros(NP48, np.int32)
_alldx[:P], _alldy[:P] = _pos[0], _pos[1]
_alldx[P:P + NN], _alldy[P:P + NN] = _neg[0], _neg[1]

NW = 32
KPW = N // NW
CK = 48
NCHUNK = KPW // CK


def _sample_body(cells_ref, out_ref):
    c = cells_ref[...]
    R = c.shape[1]
    vmax = jnp.max(c, axis=0, keepdims=True)
    ch = lax.broadcasted_iota(jnp.int32, c.shape, 0)
    k = jnp.min(jnp.where(c == vmax, ch, 64), axis=0, keepdims=True)
    r = lax.broadcasted_iota(jnp.int32, (1, R), 1)
    bidx = (r % N) // NPC
    cell = r % NPC
    cy = cell // CH
    cx = cell % CH
    i = k // 8
    j = k % 8
    xs = T + cx * 8 + j
    ys = T + cy * 8 + i
    off = bidx * HW + xs * W + ys
    z = jnp.zeros_like(off)
    out_ref[...] = jnp.concatenate([off, xs, ys, z, z, z, z, z], axis=0)


def _run_sample(det1, det2):
    crop = jnp.concatenate([det1, det2], axis=0)[:, 0, T:H - T, T:W - T]
    cells = crop.reshape(2 * B, CH, 8, CH, 8).transpose(0, 1, 3, 2, 4)
    cells = cells.reshape(2 * N, 64).T
    out = pl.pallas_call(
        _sample_body,
        out_shape=jax.ShapeDtypeStruct((8, 2 * N), jnp.int32),
    )(cells)
    return out


def _sc_body(des1v, des2v, aux, off1h, off2h, ptab,
             nallo, psco, qlto, msko, axo, ayo, s1o, dro,
             off_v, off2_v, s1_v, aux_v, idxs_v, rows_v, nall_v,
             axv, ayv, bbv, qidx_v, q2_v, pscv, qltv, mskv,
             dxt, dyt, d2_v, sem):
    wid = lax.axis_index("s") * 2 + lax.axis_index("c")
    iota = lax.iota(jnp.int32, 16)
    pltpu.sync_copy(ptab.at[0], dxt)
    pltpu.sync_copy(ptab.at[1], dyt)

    _gdn = lax.GatherDimensionNumbers(offset_dims=(), collapsed_slice_dims=(0,),
                                      start_index_map=(0,))

    def _shuf(v, s):
        return lax.gather(v, (iota ^ s)[:, None], _gdn, (1,),
                          mode=lax.GatherScatterMode.PROMISE_IN_BOUNDS)

    def _bsum(v):
        for s in (8, 4, 2, 1):
            v = v + _shuf(v, s)
        return v

    def _bmax(v):
        for s in (8, 4, 2, 1):
            v = jnp.maximum(v, _shuf(v, s))
        return v

    def _bmin(v):
        for s in (8, 4, 2, 1):
            v = jnp.minimum(v, _shuf(v, s))
        return v

    def _lane(vec, lane_idx):
        return _bsum(jnp.where(iota == lane_idx, vec, jnp.zeros_like(vec)))

    def chunk_body(ci, _):
        base = wid * KPW + ci * CK

        pltpu.sync_copy(off2h.at[pl.ds(base, CK)], off2_v)
        pltpu.async_copy(des2v.at[off2_v], d2_v, sem).wait()
        pltpu.sync_copy(d2_v, dro.at[pl.ds(base, CK)])

        pltpu.sync_copy(off1h.at[pl.ds(base, CK)], off_v)
        pltpu.async_copy(des1v.at[off_v], s1_v, sem).wait()
        pltpu.sync_copy(s1_v, s1o.at[pl.ds(base, CK)])
        pltpu.async_copy(aux.at[off_v], aux_v, sem).wait()

        def kp_body(kp, carry):
            pos_c, psc_c, ax_c, ay_c, msk_c, q1_c, qof_c = carry
            arow = aux_v[kp, pl.ds(0, 16)]
            ax_s = (_lane(arow, 0) + 0.5).astype(jnp.int32)
            ay_s = (_lane(arow, 1) + 0.5).astype(jnp.int32)
            q1_s = _lane(arow, 2)
            bb_s = _lane(arow, 4).astype(jnp.int32)
            msk_s = (jnp.where(ax_s >= 0, 1, 0) * jnp.where(ay_s >= 0, 1, 0)
                     * jnp.where(ax_s < W, 1, 0) * jnp.where(ay_s < H, 1, 0))

            waits = []
            for g in range(3):
                dxg = dxt[pl.ds(g * 16, 16)]
                dyg = dyt[pl.ds(g * 16, 16)]
                px = jnp.clip(ax_s + dxg, 0, W - 1)
                py = jnp.clip(ay_s + dyg, 0, H - 1)
                idxs_v[pl.ds(g * 16, 16)] = bb_s + py * W + px
            for g in range(3):
                waits.append(pltpu.async_copy(
                    des2v.at[idxs_v.at[pl.ds(g * 16, 16)]],
                    rows_v.at[pl.ds(g * 16, 16)], sem))
            for wd in waits:
                wd.wait()
            s1c = [s1_v[kp, pl.ds(c * 16, 16)] for c in range(8)]

            def p_body(p, sv):
                v0, v1, v2 = sv
                acc = s1c[0] * rows_v[p, pl.ds(0, 16)]
                for c in range(1, 8):
                    acc = acc + s1c[c] * rows_v[p, pl.ds(c * 16, 16)]
                dot = _bsum(acc)
                v0 = jnp.where(iota == p, dot, v0)
                v1 = jnp.where(iota == p - 16, dot, v1)
                v2 = jnp.where(iota == p - 32, dot, v2)
                return (v0, v1, v2)

            z16 = jnp.zeros((16,), jnp.float32)
            v0, v1, v2 = lax.fori_loop(0, NP48, p_body, (z16, z16, z16))
            nall_v[kp, pl.ds(0, 16)] = v0
            nall_v[kp, pl.ds(16, 16)] = v1
            nall_v[kp, pl.ds(32, 16)] = v2

            neginf = jnp.float32(-3.0e38)
            m1 = jnp.where(iota < (P - 16), v1, neginf)
            psc_s = _bmax(jnp.maximum(v0, m1))
            cand = jnp.minimum(jnp.where(v0 == psc_s, iota, 99),
                               jnp.where(m1 == psc_s, iota + 16, 99))
            pos_s = _bmin(cand)

            dx0 = dxt[pl.ds(0, 16)]
            dx1 = dxt[pl.ds(16, 16)]
            dy0 = dyt[pl.ds(0, 16)]
            dy1 = dyt[pl.ds(16, 16)]
            zi16 = jnp.zeros((16,), jnp.int32)
            dx_s = _bsum(jnp.where(iota == pos_s, dx0, zi16)
                         + jnp.where(iota + 16 == pos_s, dx1, zi16))
            dy_s = _bsum(jnp.where(iota == pos_s, dy0, zi16)
                         + jnp.where(iota + 16 == pos_s, dy1, zi16))
            qof_s = (bb_s + jnp.clip(ay_s + dy_s, 0, H - 1) * W
                     + jnp.clip(ax_s + dx_s, 0, W - 1))

            def upd(c, val):
                return tuple(jnp.where(iota == kp - 16 * g, val, c[g])
                             for g in range(3))

            return (upd(pos_c, pos_s), upd(psc_c, psc_s), upd(ax_c, ax_s),
                    upd(ay_c, ay_s), upd(msk_c, msk_s), upd(q1_c, q1_s),
                    upd(qof_c, qof_s))

        zi = jnp.zeros((16,), jnp.int32)
        zf = jnp.zeros((16,), jnp.float32)
        t3i = (zi, zi, zi)
        t3f = (zf, zf, zf)
        (pos_c, psc_c, ax_c, ay_c, msk_c, q1_c, qof_c) = lax.fori_loop(
            0, CK, kp_body, (t3i, t3f, t3i, t3i, t3i, t3f, t3i))

        for g in range(3):
            sl = pl.ds(g * 16, 16)
            qidx_v[sl] = qof_c[g]
            pscv[sl] = psc_c[g]
            axv[sl] = ax_c[g]
            ayv[sl] = ay_c[g]
            mskv[sl] = msk_c[g]

        pltpu.async_copy(aux.at[qidx_v], q2_v, sem).wait()

        def q_body(kp, qv_c):
            qrow = q2_v[kp, pl.ds(0, 16)]
            q2_s = _lane(qrow, 3)
            zf16 = jnp.zeros((16,), jnp.float32)
            q1_s = _bsum(sum(jnp.where(iota == kp - 16 * g, q1_c[g], zf16)
                             for g in range(3)))
            val = (q1_s + q2_s) * 0.5
            return tuple(jnp.where(iota == kp - 16 * g, val, qv_c[g])
                         for g in range(3))

        qv_c = lax.fori_loop(0, CK, q_body, t3f)
        for g in range(3):
            qltv[pl.ds(g * 16, 16)] = qv_c[g]

        pltpu.sync_copy(nall_v, nallo.at[pl.ds(base, CK)])
        pltpu.sync_copy(pscv, psco.at[pl.ds(base, CK)])
        pltpu.sync_copy(qltv, qlto.at[pl.ds(base, CK)])
        pltpu.sync_copy(mskv, msko.at[pl.ds(base, CK)])
        pltpu.sync_copy(axv, axo.at[pl.ds(base, CK)])
        pltpu.sync_copy(ayv, ayo.at[pl.ds(base, CK)])
        return 0

    lax.fori_loop(0, NCHUNK, chunk_body, 0)


_sc_kernel_cache = None


def _get_sc_kernel():
    global _sc_kernel_cache
    if _sc_kernel_cache is not None:
        return _sc_kernel_cache
    mesh = plsc.VectorSubcoreMesh(core_axis_name="c", subcore_axis_name="s",
                                  num_cores=2, num_subcores=16)
    _sc_kernel_cache = functools.partial(
        pl.kernel,
        compiler_params=pltpu.CompilerParams(use_tc_tiling_on_sc=False),
        out_type=(
        jax.ShapeDtypeStruct((N, NP48), jnp.float32),
        jax.ShapeDtypeStruct((N,), jnp.float32),
        jax.ShapeDtypeStruct((N,), jnp.float32),
        jax.ShapeDtypeStruct((N,), jnp.int32),
        jax.ShapeDtypeStruct((N,), jnp.int32),
        jax.ShapeDtypeStruct((N,), jnp.int32),
        jax.ShapeDtypeStruct((N, D), jnp.float32),
        jax.ShapeDtypeStruct((N, D), jnp.float32),
    ),
        mesh=mesh,
        scratch_types=[
        pltpu.VMEM((CK,), jnp.int32),
        pltpu.VMEM((CK,), jnp.int32),
        pltpu.VMEM((CK, D), jnp.float32),
        pltpu.VMEM((CK, 16), jnp.float32),
        pltpu.VMEM((CK,), jnp.int32),
        pltpu.VMEM((NP48, D), jnp.float32),
        pltpu.VMEM((CK, NP48), jnp.float32),
        pltpu.VMEM((CK,), jnp.int32),
        pltpu.VMEM((CK,), jnp.int32),
        pltpu.VMEM((CK,), jnp.int32),
        pltpu.VMEM((CK,), jnp.int32),
        pltpu.VMEM((CK, 16), jnp.float32),
        pltpu.VMEM((CK,), jnp.float32),
        pltpu.VMEM((CK,), jnp.float32),
        pltpu.VMEM((CK,), jnp.int32),
        pltpu.VMEM((NP48,), jnp.int32),
        pltpu.VMEM((NP48,), jnp.int32),
        pltpu.VMEM((CK, D), jnp.float32),
        pltpu.SemaphoreType.DMA,
        ],
    )(_sc_body)
    return _sc_kernel_cache


BM = 512


def _score_body(s13_ref, s1_ref, distr_ref, ax_ref, ay_ref, b1_ref,
                xd_ref, yd_ref, bd_ref, out_ref):
    s1 = s1_ref[...]
    dt = distr_ref[...]
    dsc = lax.dot_general(s1, dt, (((1,), (1,)), ((), ())),
                          preferred_element_type=jnp.float32)
    dx = xd_ref[...] - ax_ref[...]
    dy = yd_ref[...] - ay_ref[...]
    dis2 = dx * dx + dy * dy + jnp.where(bd_ref[...] != b1_ref[...], 9, 0)
    dsc = jnp.where(dis2 < 9, jnp.float32(0), dsc)
    out_ref[...] = jnp.concatenate([s13_ref[...][:, :1 + NN], dsc], axis=1)


def _run_scores(s13, s1, distr, ax, ay, b1, xd, yd, bd):
    grid = (N // BM,)
    return pl.pallas_call(
        _score_body,
        grid=grid,
        in_specs=[
            pl.BlockSpec((BM, 16), lambda i: (i, 0)),
            pl.BlockSpec((BM, D), lambda i: (i, 0)),
            pl.BlockSpec((N, D), lambda i: (0, 0)),
            pl.BlockSpec((BM, 1), lambda i: (i, 0)),
            pl.BlockSpec((BM, 1), lambda i: (i, 0)),
            pl.BlockSpec((BM, 1), lambda i: (i, 0)),
            pl.BlockSpec((1, N), lambda i: (0, 0)),
            pl.BlockSpec((1, N), lambda i: (0, 0)),
            pl.BlockSpec((1, N), lambda i: (0, 0)),
        ],
        out_specs=pl.BlockSpec((BM, 1 + NN + N), lambda i: (i, 0)),
        out_shape=jax.ShapeDtypeStruct((N, 1 + NN + N), jnp.float32),
    )(s13, s1, distr, ax, ay, b1, xd, yd, bd)


def kernel(des1, det1, qlt1, des2, det2, qlt2, aflow):
    des1v = des1.transpose(0, 2, 3, 1).reshape(B * HW, D)
    des2v = des2.transpose(0, 2, 3, 1).reshape(B * HW, D)
    bbase = jnp.repeat(jnp.arange(B, dtype=jnp.float32) * HW, HW)
    aux = jnp.concatenate(
        [aflow.transpose(0, 2, 3, 1).reshape(B * HW, 2),
         qlt1.reshape(B * HW, 1), qlt2.reshape(B * HW, 1),
         bbase[:, None], jnp.zeros((B * HW, 11), jnp.float32)], axis=1)
    ptab = jnp.asarray(np.stack([_alldx, _alldy]))

    samp = _run_sample(det1, det2)
    off1 = samp[0, :N]
    off2 = samp[0, N:]
    xd = samp[2, N:][None, :]
    yd = samp[1, N:][None, :]

    nall, psc, qlt, msk, ax, ay, s1, distr = _get_sc_kernel()(
        des1v, des2v, aux, off1, off2, ptab)

    b1 = jnp.repeat(jnp.arange(B, dtype=jnp.int32), NPC)
    s13 = jnp.concatenate(
        [psc[:, None], nall[:, P:P + NN],
         jnp.zeros((N, 3), jnp.float32)], axis=1)
    scores = _run_scores(s13, s1, distr, ax[:, None], ay[:, None],
                         b1[:, None], xd, yd, bd=b1[None, :])

    labels = jnp.zeros(scores.shape, dtype=bool).at[:, :1].set(True)
    mask = msk.astype(bool).reshape(B, NPC)
    return scores, labels, mask, qlt[:, None]


# speedup vs baseline: 1.1324x; 1.1324x over previous
import functools

import numpy as np
import jax
import jax.numpy as jnp
from jax import lax
from jax.experimental import pallas as pl
from jax.experimental.pallas import tpu as pltpu
from jax.experimental.pallas import tpu_sc as plsc

B, D, H, W = 8, 128, 224, 224
HW = H * W
T = 16
CH = 24
NPC = CH * CH
N = B * NPC
POS_R = 3

_pos = np.array([(i, j) for i in range(-3, 4) for j in range(-3, 4)
                 if i * i + j * j <= 9], dtype=np.int32).reshape(-1, 2).T
_neg = np.array([(i, j) for i in range(-8, 9, 2) for j in range(-8, 9, 2)
                 if 49 <= i * i + j * j <= 64], dtype=np.int32).reshape(-1, 2).T
P = _pos.shape[1]
NN = _neg.shape[1]
NP48 = 48
_alldx = np.zeros(NP48, np.int32)
_alldy = np.zeros(NP48, np.int32)
_alldx[:P], _alldy[:P] = _pos[0], _pos[1]
_alldx[P:P + NN], _alldy[P:P + NN] = _neg[0], _neg[1]

NW = 32
KPW = N // NW
CK = 48
NCHUNK = KPW // CK


def _sample_body(cells_ref, out_ref):
    c = cells_ref[...]
    R = c.shape[1]
    vmax = jnp.max(c, axis=0, keepdims=True)
    ch = lax.broadcasted_iota(jnp.int32, c.shape, 0)
    k = jnp.min(jnp.where(c == vmax, ch, 64), axis=0, keepdims=True)
    r = lax.broadcasted_iota(jnp.int32, (1, R), 1)
    bidx = (r % N) // NPC
    cell = r % NPC
    cy = cell // CH
    cx = cell % CH
    i = k // 8
    j = k % 8
    xs = T + cx * 8 + j
    ys = T + cy * 8 + i
    off = bidx * HW + xs * W + ys
    z = jnp.zeros_like(off)
    out_ref[...] = jnp.concatenate([off, xs, ys, z, z, z, z, z], axis=0)


def _run_sample(det1, det2):
    crop = jnp.concatenate([det1, det2], axis=0)[:, 0, T:H - T, T:W - T]
    cells = crop.reshape(2 * B, CH, 8, CH, 8).transpose(0, 1, 3, 2, 4)
    cells = cells.reshape(2 * N, 64).T
    out = pl.pallas_call(
        _sample_body,
        out_shape=jax.ShapeDtypeStruct((8, 2 * N), jnp.int32),
    )(cells)
    return out


def _sc_body(des1v, des2v, aux, off1h, off2h, ptab,
             nallo, psco, qlto, msko, axo, ayo, s1o, dro,
             off_v, off2_v, s1_v, aux_v, nidx_v, rows2_v, nall_v,
             axv, ayv, bbv, qidx_v, q2_v, pscv, qltv, mskv,
             dxt, dyt, d2_v, sem, sema, semb):
    wid = lax.axis_index("s") * 2 + lax.axis_index("c")
    iota = lax.iota(jnp.int32, 16)
    pltpu.sync_copy(ptab.at[0], dxt)
    pltpu.sync_copy(ptab.at[1], dyt)

    _gdn = lax.GatherDimensionNumbers(offset_dims=(), collapsed_slice_dims=(0,),
                                      start_index_map=(0,))

    def _shuf(v, s):
        return lax.gather(v, (iota ^ s)[:, None], _gdn, (1,),
                          mode=lax.GatherScatterMode.PROMISE_IN_BOUNDS)

    def _bsum(v):
        for s in (8, 4, 2, 1):
            v = v + _shuf(v, s)
        return v

    def _bmax(v):
        for s in (8, 4, 2, 1):
            v = jnp.maximum(v, _shuf(v, s))
        return v

    def _bmin(v):
        for s in (8, 4, 2, 1):
            v = jnp.minimum(v, _shuf(v, s))
        return v

    def _lane(vec, lane_idx):
        return _bsum(jnp.where(iota == lane_idx, vec, jnp.zeros_like(vec)))

    def chunk_body(ci, _):
        base = wid * KPW + ci * CK

        pltpu.sync_copy(off2h.at[pl.ds(base, CK)], off2_v)
        pltpu.async_copy(des2v.at[off2_v], d2_v, sem).wait()
        pltpu.sync_copy(d2_v, dro.at[pl.ds(base, CK)])

        pltpu.sync_copy(off1h.at[pl.ds(base, CK)], off_v)
        pltpu.async_copy(des1v.at[off_v], s1_v, sem).wait()
        pltpu.sync_copy(s1_v, s1o.at[pl.ds(base, CK)])
        pltpu.async_copy(aux.at[off_v], aux_v, sem).wait()

        def idx_body(kp, _):
            arow = aux_v[kp, pl.ds(0, 16)]
            ax_s = (_lane(arow, 0) + 0.5).astype(jnp.int32)
            ay_s = (_lane(arow, 1) + 0.5).astype(jnp.int32)
            bb_s = _lane(arow, 4).astype(jnp.int32)
            for g in range(3):
                dxg = dxt[pl.ds(g * 16, 16)]
                dyg = dyt[pl.ds(g * 16, 16)]
                px = jnp.clip(ax_s + dxg, 0, W - 1)
                py = jnp.clip(ay_s + dyg, 0, H - 1)
                nidx_v[kp, pl.ds(g * 16, 16)] = bb_s + py * W + px
            return 0

        lax.fori_loop(0, CK, idx_body, 0)

        sems = (sema, semb)

        def issue(kp):
            @pl.when(kp % 2 == 0)
            def _():
                pltpu.async_copy(des2v.at[nidx_v.at[kp]], rows2_v.at[0], sema)

            @pl.when(kp % 2 == 1)
            def _():
                pltpu.async_copy(des2v.at[nidx_v.at[kp]], rows2_v.at[1], semb)

        def drain(kp):
            @pl.when(kp % 2 == 0)
            def _():
                pltpu.make_async_copy(des2v.at[pl.ds(0, NP48)],
                                      rows2_v.at[0], sema).wait()

            @pl.when(kp % 2 == 1)
            def _():
                pltpu.make_async_copy(des2v.at[pl.ds(0, NP48)],
                                      rows2_v.at[1], semb).wait()

        issue(0)

        def kp_body(kp, carry):
            pos_c, psc_c, ax_c, ay_c, msk_c, q1_c, qof_c = carry
            arow = aux_v[kp, pl.ds(0, 16)]
            ax_s = (_lane(arow, 0) + 0.5).astype(jnp.int32)
            ay_s = (_lane(arow, 1) + 0.5).astype(jnp.int32)
            q1_s = _lane(arow, 2)
            bb_s = _lane(arow, 4).astype(jnp.int32)
            msk_s = (jnp.where(ax_s >= 0, 1, 0) * jnp.where(ay_s >= 0, 1, 0)
                     * jnp.where(ax_s < W, 1, 0) * jnp.where(ay_s < H, 1, 0))

            @pl.when(kp + 1 < CK)
            def _():
                issue(kp + 1)

            drain(kp)
            par = kp % 2
            s1c = [s1_v[kp, pl.ds(c * 16, 16)] for c in range(8)]

            def p_body(p, sv):
                v0, v1, v2 = sv
                acc = s1c[0] * rows2_v[par, p, pl.ds(0, 16)]
                for c in range(1, 8):
                    acc = acc + s1c[c] * rows2_v[par, p, pl.ds(c * 16, 16)]
                dot = _bsum(acc)
                v0 = jnp.where(iota == p, dot, v0)
                v1 = jnp.where(iota == p - 16, dot, v1)
                v2 = jnp.where(iota == p - 32, dot, v2)
                return (v0, v1, v2)

            z16 = jnp.zeros((16,), jnp.float32)
            v0, v1, v2 = lax.fori_loop(0, NP48, p_body, (z16, z16, z16))
            nall_v[kp, pl.ds(0, 16)] = v0
            nall_v[kp, pl.ds(16, 16)] = v1
            nall_v[kp, pl.ds(32, 16)] = v2

            neginf = jnp.float32(-3.0e38)
            m1 = jnp.where(iota < (P - 16), v1, neginf)
            psc_s = _bmax(jnp.maximum(v0, m1))
            cand = jnp.minimum(jnp.where(v0 == psc_s, iota, 99),
                               jnp.where(m1 == psc_s, iota + 16, 99))
            pos_s = _bmin(cand)

            dx0 = dxt[pl.ds(0, 16)]
            dx1 = dxt[pl.ds(16, 16)]
            dy0 = dyt[pl.ds(0, 16)]
            dy1 = dyt[pl.ds(16, 16)]
            zi16 = jnp.zeros((16,), jnp.int32)
            dx_s = _bsum(jnp.where(iota == pos_s, dx0, zi16)
                         + jnp.where(iota + 16 == pos_s, dx1, zi16))
            dy_s = _bsum(jnp.where(iota == pos_s, dy0, zi16)
                         + jnp.where(iota + 16 == pos_s, dy1, zi16))
            qof_s = (bb_s + jnp.clip(ay_s + dy_s, 0, H - 1) * W
                     + jnp.clip(ax_s + dx_s, 0, W - 1))

            def upd(c, val):
                return tuple(jnp.where(iota == kp - 16 * g, val, c[g])
                             for g in range(3))

            return (upd(pos_c, pos_s), upd(psc_c, psc_s), upd(ax_c, ax_s),
                    upd(ay_c, ay_s), upd(msk_c, msk_s), upd(q1_c, q1_s),
                    upd(qof_c, qof_s))

        zi = jnp.zeros((16,), jnp.int32)
        zf = jnp.zeros((16,), jnp.float32)
        t3i = (zi, zi, zi)
        t3f = (zf, zf, zf)
        (pos_c, psc_c, ax_c, ay_c, msk_c, q1_c, qof_c) = lax.fori_loop(
            0, CK, kp_body, (t3i, t3f, t3i, t3i, t3i, t3f, t3i))

        for g in range(3):
            sl = pl.ds(g * 16, 16)
            qidx_v[sl] = qof_c[g]
            pscv[sl] = psc_c[g]
            axv[sl] = ax_c[g]
            ayv[sl] = ay_c[g]
            mskv[sl] = msk_c[g]

        pltpu.async_copy(aux.at[qidx_v], q2_v, sem).wait()

        def q_body(kp, qv_c):
            qrow = q2_v[kp, pl.ds(0, 16)]
            q2_s = _lane(qrow, 3)
            zf16 = jnp.zeros((16,), jnp.float32)
            q1_s = _bsum(sum(jnp.where(iota == kp - 16 * g, q1_c[g], zf16)
                             for g in range(3)))
            val = (q1_s + q2_s) * 0.5
            return tuple(jnp.where(iota == kp - 16 * g, val, qv_c[g])
                         for g in range(3))

        qv_c = lax.fori_loop(0, CK, q_body, t3f)
        for g in range(3):
            qltv[pl.ds(g * 16, 16)] = qv_c[g]

        pltpu.sync_copy(nall_v, nallo.at[pl.ds(base, CK)])
        pltpu.sync_copy(pscv, psco.at[pl.ds(base, CK)])
        pltpu.sync_copy(qltv, qlto.at[pl.ds(base, CK)])
        pltpu.sync_copy(mskv, msko.at[pl.ds(base, CK)])
        pltpu.sync_copy(axv, axo.at[pl.ds(base, CK)])
        pltpu.sync_copy(ayv, ayo.at[pl.ds(base, CK)])
        return 0

    lax.fori_loop(0, NCHUNK, chunk_body, 0)


_sc_kernel_cache = None


def _get_sc_kernel():
    global _sc_kernel_cache
    if _sc_kernel_cache is not None:
        return _sc_kernel_cache
    mesh = plsc.VectorSubcoreMesh(core_axis_name="c", subcore_axis_name="s",
                                  num_cores=2, num_subcores=16)
    _sc_kernel_cache = functools.partial(
        pl.kernel,
        compiler_params=pltpu.CompilerParams(use_tc_tiling_on_sc=False),
        out_type=(
        jax.ShapeDtypeStruct((N, NP48), jnp.float32),
        jax.ShapeDtypeStruct((N,), jnp.float32),
        jax.ShapeDtypeStruct((N,), jnp.float32),
        jax.ShapeDtypeStruct((N,), jnp.int32),
        jax.ShapeDtypeStruct((N,), jnp.int32),
        jax.ShapeDtypeStruct((N,), jnp.int32),
        jax.ShapeDtypeStruct((N, D), jnp.float32),
        jax.ShapeDtypeStruct((N, D), jnp.float32),
    ),
        mesh=mesh,
        scratch_types=[
        pltpu.VMEM((CK,), jnp.int32),
        pltpu.VMEM((CK,), jnp.int32),
        pltpu.VMEM((CK, D), jnp.float32),
        pltpu.VMEM((CK, 16), jnp.float32),
        pltpu.VMEM((CK, NP48), jnp.int32),
        pltpu.VMEM((2, NP48, D), jnp.float32),
        pltpu.VMEM((CK, NP48), jnp.float32),
        pltpu.VMEM((CK,), jnp.int32),
        pltpu.VMEM((CK,), jnp.int32),
        pltpu.VMEM((CK,), jnp.int32),
        pltpu.VMEM((CK,), jnp.int32),
        pltpu.VMEM((CK, 16), jnp.float32),
        pltpu.VMEM((CK,), jnp.float32),
        pltpu.VMEM((CK,), jnp.float32),
        pltpu.VMEM((CK,), jnp.int32),
        pltpu.VMEM((NP48,), jnp.int32),
        pltpu.VMEM((NP48,), jnp.int32),
        pltpu.VMEM((CK, D), jnp.float32),
        pltpu.SemaphoreType.DMA,
        pltpu.SemaphoreType.DMA,
        pltpu.SemaphoreType.DMA,
        ],
    )(_sc_body)
    return _sc_kernel_cache


BM = 512


def _score_body(s13_ref, s1_ref, distr_ref, ax_ref, ay_ref, b1_ref,
                xd_ref, yd_ref, bd_ref, out_ref):
    s1 = s1_ref[...]
    dt = distr_ref[...]
    dsc = lax.dot_general(s1, dt, (((1,), (1,)), ((), ())),
                          preferred_element_type=jnp.float32)
    dx = xd_ref[...] - ax_ref[...]
    dy = yd_ref[...] - ay_ref[...]
    dis2 = dx * dx + dy * dy + jnp.where(bd_ref[...] != b1_ref[...], 9, 0)
    dsc = jnp.where(dis2 < 9, jnp.float32(0), dsc)
    out_ref[...] = jnp.concatenate([s13_ref[...][:, :1 + NN], dsc], axis=1)


def _run_scores(s13, s1, distr, ax, ay, b1, xd, yd, bd):
    grid = (N // BM,)
    return pl.pallas_call(
        _score_body,
        grid=grid,
        in_specs=[
            pl.BlockSpec((BM, 16), lambda i: (i, 0)),
            pl.BlockSpec((BM, D), lambda i: (i, 0)),
            pl.BlockSpec((N, D), lambda i: (0, 0)),
            pl.BlockSpec((BM, 1), lambda i: (i, 0)),
            pl.BlockSpec((BM, 1), lambda i: (i, 0)),
            pl.BlockSpec((BM, 1), lambda i: (i, 0)),
            pl.BlockSpec((1, N), lambda i: (0, 0)),
            pl.BlockSpec((1, N), lambda i: (0, 0)),
            pl.BlockSpec((1, N), lambda i: (0, 0)),
        ],
        out_specs=pl.BlockSpec((BM, 1 + NN + N), lambda i: (i, 0)),
        out_shape=jax.ShapeDtypeStruct((N, 1 + NN + N), jnp.float32),
    )(s13, s1, distr, ax, ay, b1, xd, yd, bd)


def kernel(des1, det1, qlt1, des2, det2, qlt2, aflow):
    des1v = des1.transpose(0, 2, 3, 1).reshape(B * HW, D)
    des2v = des2.transpose(0, 2, 3, 1).reshape(B * HW, D)
    bbase = jnp.repeat(jnp.arange(B, dtype=jnp.float32) * HW, HW)
    aux = jnp.concatenate(
        [aflow.transpose(0, 2, 3, 1).reshape(B * HW, 2),
         qlt1.reshape(B * HW, 1), qlt2.reshape(B * HW, 1),
         bbase[:, None], jnp.zeros((B * HW, 11), jnp.float32)], axis=1)
    ptab = jnp.asarray(np.stack([_alldx, _alldy]))

    samp = _run_sample(det1, det2)
    off1 = samp[0, :N]
    off2 = samp[0, N:]
    xd = samp[2, N:][None, :]
    yd = samp[1, N:][None, :]

    nall, psc, qlt, msk, ax, ay, s1, distr = _get_sc_kernel()(
        des1v, des2v, aux, off1, off2, ptab)

    b1 = jnp.repeat(jnp.arange(B, dtype=jnp.int32), NPC)
    s13 = jnp.concatenate(
        [psc[:, None], nall[:, P:P + NN],
         jnp.zeros((N, 3), jnp.float32)], axis=1)
    scores = _run_scores(s13, s1, distr, ax[:, None], ay[:, None],
                         b1[:, None], xd, yd, bd=b1[None, :])

    labels = jnp.zeros(scores.shape, dtype=bool).at[:, :1].set(True)
    mask = msk.astype(bool).reshape(B, NPC)
    return scores, labels, mask, qlt[:, None]


# speedup vs baseline: 1.2185x; 1.0760x over previous
import functools

import numpy as np
import jax
import jax.numpy as jnp
from jax import lax
from jax.experimental import pallas as pl
from jax.experimental.pallas import tpu as pltpu
from jax.experimental.pallas import tpu_sc as plsc

B, D, H, W = 8, 128, 224, 224
HW = H * W
T = 16
CH = 24
NPC = CH * CH
N = B * NPC
POS_R = 3

_pos = np.array([(i, j) for i in range(-3, 4) for j in range(-3, 4)
                 if i * i + j * j <= 9], dtype=np.int32).reshape(-1, 2).T
_neg = np.array([(i, j) for i in range(-8, 9, 2) for j in range(-8, 9, 2)
                 if 49 <= i * i + j * j <= 64], dtype=np.int32).reshape(-1, 2).T
P = _pos.shape[1]
NN = _neg.shape[1]
NP48 = 48
_alldx = np.zeros(NP48, np.int32)
_alldy = np.zeros(NP48, np.int32)
_alldx[:P], _alldy[:P] = _pos[0], _pos[1]
_alldx[P:P + NN], _alldy[P:P + NN] = _neg[0], _neg[1]

NBUF = 4
NW = 32
KPW = N // NW
CK = 48
NCHUNK = KPW // CK


def _sample_body(cells_ref, out_ref):
    c = cells_ref[...]
    R = c.shape[1]
    vmax = jnp.max(c, axis=0, keepdims=True)
    ch = lax.broadcasted_iota(jnp.int32, c.shape, 0)
    k = jnp.min(jnp.where(c == vmax, ch, 64), axis=0, keepdims=True)
    r = lax.broadcasted_iota(jnp.int32, (1, R), 1)
    bidx = (r % N) // NPC
    cell = r % NPC
    cy = cell // CH
    cx = cell % CH
    i = k // 8
    j = k % 8
    xs = T + cx * 8 + j
    ys = T + cy * 8 + i
    off = bidx * HW + xs * W + ys
    z = jnp.zeros_like(off)
    out_ref[...] = jnp.concatenate([off, xs, ys, z, z, z, z, z], axis=0)


def _run_sample(det1, det2):
    crop = jnp.concatenate([det1, det2], axis=0)[:, 0, T:H - T, T:W - T]
    cells = crop.reshape(2 * B, CH, 8, CH, 8).transpose(0, 1, 3, 2, 4)
    cells = cells.reshape(2 * N, 64).T
    out = pl.pallas_call(
        _sample_body,
        out_shape=jax.ShapeDtypeStruct((8, 2 * N), jnp.int32),
    )(cells)
    return out


def _sc_body(des1v, des2v, aux, off1h, off2h, ptab,
             nallo, psco, qlto, msko, axo, ayo, s1o, dro,
             off_v, off2_v, s1_v, aux_v, nidx_v, rows2_v, nall_v,
             axv, ayv, bbv, qidx_v, q2_v, pscv, qltv, mskv,
             dxt, dyt, d2_v, sem, sema, semb, semc, semd):
    wid = lax.axis_index("s") * 2 + lax.axis_index("c")
    iota = lax.iota(jnp.int32, 16)
    pltpu.sync_copy(ptab.at[0], dxt)
    pltpu.sync_copy(ptab.at[1], dyt)

    _gdn = lax.GatherDimensionNumbers(offset_dims=(), collapsed_slice_dims=(0,),
                                      start_index_map=(0,))

    def _shuf(v, s):
        return lax.gather(v, (iota ^ s)[:, None], _gdn, (1,),
                          mode=lax.GatherScatterMode.PROMISE_IN_BOUNDS)

    def _bsum(v):
        for s in (8, 4, 2, 1):
            v = v + _shuf(v, s)
        return v

    def _bmax(v):
        for s in (8, 4, 2, 1):
            v = jnp.maximum(v, _shuf(v, s))
        return v

    def _bmin(v):
        for s in (8, 4, 2, 1):
            v = jnp.minimum(v, _shuf(v, s))
        return v

    def _lane(vec, lane_idx):
        return _bsum(jnp.where(iota == lane_idx, vec, jnp.zeros_like(vec)))

    def chunk_body(ci, _):
        base = wid * KPW + ci * CK

        pltpu.sync_copy(off2h.at[pl.ds(base, CK)], off2_v)
        pltpu.async_copy(des2v.at[off2_v], d2_v, sem).wait()
        pltpu.sync_copy(d2_v, dro.at[pl.ds(base, CK)])

        pltpu.sync_copy(off1h.at[pl.ds(base, CK)], off_v)
        pltpu.async_copy(des1v.at[off_v], s1_v, sem).wait()
        pltpu.sync_copy(s1_v, s1o.at[pl.ds(base, CK)])
        pltpu.async_copy(aux.at[off_v], aux_v, sem).wait()

        def idx_body(kp, _):
            arow = aux_v[kp, pl.ds(0, 16)]
            ax_s = (_lane(arow, 0) + 0.5).astype(jnp.int32)
            ay_s = (_lane(arow, 1) + 0.5).astype(jnp.int32)
            bb_s = _lane(arow, 4).astype(jnp.int32)
            for g in range(3):
                dxg = dxt[pl.ds(g * 16, 16)]
                dyg = dyt[pl.ds(g * 16, 16)]
                px = jnp.clip(ax_s + dxg, 0, W - 1)
                py = jnp.clip(ay_s + dyg, 0, H - 1)
                nidx_v[kp, pl.ds(g * 16, 16)] = bb_s + py * W + px
            return 0

        lax.fori_loop(0, CK, idx_body, 0)

        sems = (sema, semb, semc, semd)

        def issue(kp):
            for b in range(NBUF):
                @pl.when(kp % NBUF == b)
                def _(b=b):
                    pltpu.async_copy(des2v.at[nidx_v.at[kp]],
                                     rows2_v.at[b], sems[b])

        def drain(kp):
            for b in range(NBUF):
                @pl.when(kp % NBUF == b)
                def _(b=b):
                    pltpu.make_async_copy(des2v.at[pl.ds(0, NP48)],
                                          rows2_v.at[b], sems[b]).wait()

        for k0 in range(NBUF - 1):
            issue(k0)

        def kp_body(kp, carry):
            pos_c, psc_c, ax_c, ay_c, msk_c, q1_c, qof_c = carry
            arow = aux_v[kp, pl.ds(0, 16)]
            ax_s = (_lane(arow, 0) + 0.5).astype(jnp.int32)
            ay_s = (_lane(arow, 1) + 0.5).astype(jnp.int32)
            q1_s = _lane(arow, 2)
            bb_s = _lane(arow, 4).astype(jnp.int32)
            msk_s = (jnp.where(ax_s >= 0, 1, 0) * jnp.where(ay_s >= 0, 1, 0)
                     * jnp.where(ax_s < W, 1, 0) * jnp.where(ay_s < H, 1, 0))

            @pl.when(kp + (NBUF - 1) < CK)
            def _():
                issue(kp + (NBUF - 1))

            drain(kp)
            par = kp % NBUF
            s1c = [s1_v[kp, pl.ds(c * 16, 16)] for c in range(8)]

            def p_body(p, sv):
                v0, v1, v2 = sv
                acc = s1c[0] * rows2_v[par, p, pl.ds(0, 16)]
                for c in range(1, 8):
                    acc = acc + s1c[c] * rows2_v[par, p, pl.ds(c * 16, 16)]
                dot = _bsum(acc)
                v0 = jnp.where(iota == p, dot, v0)
                v1 = jnp.where(iota == p - 16, dot, v1)
                v2 = jnp.where(iota == p - 32, dot, v2)
                return (v0, v1, v2)

            z16 = jnp.zeros((16,), jnp.float32)
            v0, v1, v2 = lax.fori_loop(0, NP48, p_body, (z16, z16, z16))
            nall_v[kp, pl.ds(0, 16)] = v0
            nall_v[kp, pl.ds(16, 16)] = v1
            nall_v[kp, pl.ds(32, 16)] = v2

            neginf = jnp.float32(-3.0e38)
            m1 = jnp.where(iota < (P - 16), v1, neginf)
            psc_s = _bmax(jnp.maximum(v0, m1))
            cand = jnp.minimum(jnp.where(v0 == psc_s, iota, 99),
                               jnp.where(m1 == psc_s, iota + 16, 99))
            pos_s = _bmin(cand)

            dx0 = dxt[pl.ds(0, 16)]
            dx1 = dxt[pl.ds(16, 16)]
            dy0 = dyt[pl.ds(0, 16)]
            dy1 = dyt[pl.ds(16, 16)]
            zi16 = jnp.zeros((16,), jnp.int32)
            dx_s = _bsum(jnp.where(iota == pos_s, dx0, zi16)
                         + jnp.where(iota + 16 == pos_s, dx1, zi16))
            dy_s = _bsum(jnp.where(iota == pos_s, dy0, zi16)
                         + jnp.where(iota + 16 == pos_s, dy1, zi16))
            qof_s = (bb_s + jnp.clip(ay_s + dy_s, 0, H - 1) * W
                     + jnp.clip(ax_s + dx_s, 0, W - 1))

            def upd(c, val):
                return tuple(jnp.where(iota == kp - 16 * g, val, c[g])
                             for g in range(3))

            return (upd(pos_c, pos_s), upd(psc_c, psc_s), upd(ax_c, ax_s),
                    upd(ay_c, ay_s), upd(msk_c, msk_s), upd(q1_c, q1_s),
                    upd(qof_c, qof_s))

        zi = jnp.zeros((16,), jnp.int32)
        zf = jnp.zeros((16,), jnp.float32)
        t3i = (zi, zi, zi)
        t3f = (zf, zf, zf)
        (pos_c, psc_c, ax_c, ay_c, msk_c, q1_c, qof_c) = lax.fori_loop(
            0, CK, kp_body, (t3i, t3f, t3i, t3i, t3i, t3f, t3i))

        for g in range(3):
            sl = pl.ds(g * 16, 16)
            qidx_v[sl] = qof_c[g]
            pscv[sl] = psc_c[g]
            axv[sl] = ax_c[g]
            ayv[sl] = ay_c[g]
            mskv[sl] = msk_c[g]

        pltpu.async_copy(aux.at[qidx_v], q2_v, sem).wait()

        def q_body(kp, qv_c):
            qrow = q2_v[kp, pl.ds(0, 16)]
            q2_s = _lane(qrow, 3)
            zf16 = jnp.zeros((16,), jnp.float32)
            q1_s = _bsum(sum(jnp.where(iota == kp - 16 * g, q1_c[g], zf16)
                             for g in range(3)))
            val = (q1_s + q2_s) * 0.5
            return tuple(jnp.where(iota == kp - 16 * g, val, qv_c[g])
                         for g in range(3))

        qv_c = lax.fori_loop(0, CK, q_body, t3f)
        for g in range(3):
            qltv[pl.ds(g * 16, 16)] = qv_c[g]

        pltpu.sync_copy(nall_v, nallo.at[pl.ds(base, CK)])
        pltpu.sync_copy(pscv, psco.at[pl.ds(base, CK)])
        pltpu.sync_copy(qltv, qlto.at[pl.ds(base, CK)])
        pltpu.sync_copy(mskv, msko.at[pl.ds(base, CK)])
        pltpu.sync_copy(axv, axo.at[pl.ds(base, CK)])
        pltpu.sync_copy(ayv, ayo.at[pl.ds(base, CK)])
        return 0

    lax.fori_loop(0, NCHUNK, chunk_body, 0)


_sc_kernel_cache = None


def _get_sc_kernel():
    global _sc_kernel_cache
    if _sc_kernel_cache is not None:
        return _sc_kernel_cache
    mesh = plsc.VectorSubcoreMesh(core_axis_name="c", subcore_axis_name="s",
                                  num_cores=2, num_subcores=16)
    _sc_kernel_cache = functools.partial(
        pl.kernel,
        compiler_params=pltpu.CompilerParams(use_tc_tiling_on_sc=False),
        out_type=(
        jax.ShapeDtypeStruct((N, NP48), jnp.float32),
        jax.ShapeDtypeStruct((N,), jnp.float32),
        jax.ShapeDtypeStruct((N,), jnp.float32),
        jax.ShapeDtypeStruct((N,), jnp.int32),
        jax.ShapeDtypeStruct((N,), jnp.int32),
        jax.ShapeDtypeStruct((N,), jnp.int32),
        jax.ShapeDtypeStruct((N, D), jnp.float32),
        jax.ShapeDtypeStruct((N, D), jnp.float32),
    ),
        mesh=mesh,
        scratch_types=[
        pltpu.VMEM((CK,), jnp.int32),
        pltpu.VMEM((CK,), jnp.int32),
        pltpu.VMEM((CK, D), jnp.float32),
        pltpu.VMEM((CK, 16), jnp.float32),
        pltpu.VMEM((CK, NP48), jnp.int32),
        pltpu.VMEM((4, NP48, D), jnp.float32),
        pltpu.VMEM((CK, NP48), jnp.float32),
        pltpu.VMEM((CK,), jnp.int32),
        pltpu.VMEM((CK,), jnp.int32),
        pltpu.VMEM((CK,), jnp.int32),
        pltpu.VMEM((CK,), jnp.int32),
        pltpu.VMEM((CK, 16), jnp.float32),
        pltpu.VMEM((CK,), jnp.float32),
        pltpu.VMEM((CK,), jnp.float32),
        pltpu.VMEM((CK,), jnp.int32),
        pltpu.VMEM((NP48,), jnp.int32),
        pltpu.VMEM((NP48,), jnp.int32),
        pltpu.VMEM((CK, D), jnp.float32),
        pltpu.SemaphoreType.DMA,
        pltpu.SemaphoreType.DMA,
        pltpu.SemaphoreType.DMA,
        pltpu.SemaphoreType.DMA,
        pltpu.SemaphoreType.DMA,
        ],
    )(_sc_body)
    return _sc_kernel_cache


BM = 512


def _score_body(s13_ref, s1_ref, distr_ref, ax_ref, ay_ref, b1_ref,
                xd_ref, yd_ref, bd_ref, out_ref):
    s1 = s1_ref[...]
    dt = distr_ref[...]
    dsc = lax.dot_general(s1, dt, (((1,), (1,)), ((), ())),
                          preferred_element_type=jnp.float32)
    dx = xd_ref[...] - ax_ref[...]
    dy = yd_ref[...] - ay_ref[...]
    dis2 = dx * dx + dy * dy + jnp.where(bd_ref[...] != b1_ref[...], 9, 0)
    dsc = jnp.where(dis2 < 9, jnp.float32(0), dsc)
    out_ref[...] = jnp.concatenate([s13_ref[...][:, :1 + NN], dsc], axis=1)


def _run_scores(s13, s1, distr, ax, ay, b1, xd, yd, bd):
    grid = (N // BM,)
    return pl.pallas_call(
        _score_body,
        grid=grid,
        in_specs=[
            pl.BlockSpec((BM, 16), lambda i: (i, 0)),
            pl.BlockSpec((BM, D), lambda i: (i, 0)),
            pl.BlockSpec((N, D), lambda i: (0, 0)),
            pl.BlockSpec((BM, 1), lambda i: (i, 0)),
            pl.BlockSpec((BM, 1), lambda i: (i, 0)),
            pl.BlockSpec((BM, 1), lambda i: (i, 0)),
            pl.BlockSpec((1, N), lambda i: (0, 0)),
            pl.BlockSpec((1, N), lambda i: (0, 0)),
            pl.BlockSpec((1, N), lambda i: (0, 0)),
        ],
        out_specs=pl.BlockSpec((BM, 1 + NN + N), lambda i: (i, 0)),
        out_shape=jax.ShapeDtypeStruct((N, 1 + NN + N), jnp.float32),
    )(s13, s1, distr, ax, ay, b1, xd, yd, bd)


def kernel(des1, det1, qlt1, des2, det2, qlt2, aflow):
    des1v = des1.transpose(0, 2, 3, 1).reshape(B * HW, D)
    des2v = des2.transpose(0, 2, 3, 1).reshape(B * HW, D)
    bbase = jnp.repeat(jnp.arange(B, dtype=jnp.float32) * HW, HW)
    aux = jnp.concatenate(
        [aflow.transpose(0, 2, 3, 1).reshape(B * HW, 2),
         qlt1.reshape(B * HW, 1), qlt2.reshape(B * HW, 1),
         bbase[:, None], jnp.zeros((B * HW, 11), jnp.float32)], axis=1)
    ptab = jnp.asarray(np.stack([_alldx, _alldy]))

    samp = _run_sample(det1, det2)
    off1 = samp[0, :N]
    off2 = samp[0, N:]
    xd = samp[2, N:][None, :]
    yd = samp[1, N:][None, :]

    nall, psc, qlt, msk, ax, ay, s1, distr = _get_sc_kernel()(
        des1v, des2v, aux, off1, off2, ptab)

    b1 = jnp.repeat(jnp.arange(B, dtype=jnp.int32), NPC)
    s13 = jnp.concatenate(
        [psc[:, None], nall[:, P:P + NN],
         jnp.zeros((N, 3), jnp.float32)], axis=1)
    scores = _run_scores(s13, s1, distr, ax[:, None], ay[:, None],
                         b1[:, None], xd, yd, bd=b1[None, :])

    labels = jnp.zeros(scores.shape, dtype=bool).at[:, :1].set(True)
    mask = msk.astype(bool).reshape(B, NPC)
    return scores, labels, mask, qlt[:, None]


# speedup vs baseline: 1.2403x; 1.0179x over previous
import functools

import numpy as np
import jax
import jax.numpy as jnp
from jax import lax
from jax.experimental import pallas as pl
from jax.experimental.pallas import tpu as pltpu
from jax.experimental.pallas import tpu_sc as plsc

B, D, H, W = 8, 128, 224, 224
HW = H * W
T = 16
CH = 24
NPC = CH * CH
N = B * NPC
POS_R = 3

_pos = np.array([(i, j) for i in range(-3, 4) for j in range(-3, 4)
                 if i * i + j * j <= 9], dtype=np.int32).reshape(-1, 2).T
_neg = np.array([(i, j) for i in range(-8, 9, 2) for j in range(-8, 9, 2)
                 if 49 <= i * i + j * j <= 64], dtype=np.int32).reshape(-1, 2).T
P = _pos.shape[1]
NN = _neg.shape[1]
NP48 = 48
_alldx = np.zeros(NP48, np.int32)
_alldy = np.zeros(NP48, np.int32)
_alldx[:P], _alldy[:P] = _pos[0], _pos[1]
_alldx[P:P + NN], _alldy[P:P + NN] = _neg[0], _neg[1]

NBUF = 8
NW = 32
KPW = N // NW
CK = 48
NCHUNK = KPW // CK


def _sample_body(cells_ref, out_ref):
    c = cells_ref[...]
    R = c.shape[1]
    vmax = jnp.max(c, axis=0, keepdims=True)
    ch = lax.broadcasted_iota(jnp.int32, c.shape, 0)
    k = jnp.min(jnp.where(c == vmax, ch, 64), axis=0, keepdims=True)
    r = lax.broadcasted_iota(jnp.int32, (1, R), 1)
    bidx = (r % N) // NPC
    cell = r % NPC
    cy = cell // CH
    cx = cell % CH
    i = k // 8
    j = k % 8
    xs = T + cx * 8 + j
    ys = T + cy * 8 + i
    off = bidx * HW + xs * W + ys
    z = jnp.zeros_like(off)
    out_ref[...] = jnp.concatenate([off, xs, ys, z, z, z, z, z], axis=0)


def _run_sample(det1, det2):
    crop = jnp.concatenate([det1, det2], axis=0)[:, 0, T:H - T, T:W - T]
    cells = crop.reshape(2 * B, CH, 8, CH, 8).transpose(0, 1, 3, 2, 4)
    cells = cells.reshape(2 * N, 64).T
    out = pl.pallas_call(
        _sample_body,
        out_shape=jax.ShapeDtypeStruct((8, 2 * N), jnp.int32),
    )(cells)
    return out


def _sc_body(des1v, des2v, aux, off1h, off2h, ptab,
             nallo, psco, qlto, msko, axo, ayo, s1o, dro,
             off_v, off2_v, s1_v, aux_v, nidx_v, rows2_v, nall_v,
             axv, ayv, bbv, qidx_v, q2_v, pscv, qltv, mskv,
             dxt, dyt, d2_v, sem, sema, semb, semc, semd, seme, semf, semg, semh):
    wid = lax.axis_index("s") * 2 + lax.axis_index("c")
    iota = lax.iota(jnp.int32, 16)
    pltpu.sync_copy(ptab.at[0], dxt)
    pltpu.sync_copy(ptab.at[1], dyt)

    _gdn = lax.GatherDimensionNumbers(offset_dims=(), collapsed_slice_dims=(0,),
                                      start_index_map=(0,))

    def _shuf(v, s):
        return lax.gather(v, (iota ^ s)[:, None], _gdn, (1,),
                          mode=lax.GatherScatterMode.PROMISE_IN_BOUNDS)

    def _bsum(v):
        for s in (8, 4, 2, 1):
            v = v + _shuf(v, s)
        return v

    def _bmax(v):
        for s in (8, 4, 2, 1):
            v = jnp.maximum(v, _shuf(v, s))
        return v

    def _bmin(v):
        for s in (8, 4, 2, 1):
            v = jnp.minimum(v, _shuf(v, s))
        return v

    def _lane(vec, lane_idx):
        return _bsum(jnp.where(iota == lane_idx, vec, jnp.zeros_like(vec)))

    def chunk_body(ci, _):
        base = wid * KPW + ci * CK

        pltpu.sync_copy(off2h.at[pl.ds(base, CK)], off2_v)
        pltpu.async_copy(des2v.at[off2_v], d2_v, sem).wait()
        pltpu.sync_copy(d2_v, dro.at[pl.ds(base, CK)])

        pltpu.sync_copy(off1h.at[pl.ds(base, CK)], off_v)
        pltpu.async_copy(des1v.at[off_v], s1_v, sem).wait()
        pltpu.sync_copy(s1_v, s1o.at[pl.ds(base, CK)])
        pltpu.async_copy(aux.at[off_v], aux_v, sem).wait()

        def idx_body(kp, _):
            arow = aux_v[kp, pl.ds(0, 16)]
            ax_s = (_lane(arow, 0) + 0.5).astype(jnp.int32)
            ay_s = (_lane(arow, 1) + 0.5).astype(jnp.int32)
            bb_s = _lane(arow, 4).astype(jnp.int32)
            for g in range(3):
                dxg = dxt[pl.ds(g * 16, 16)]
                dyg = dyt[pl.ds(g * 16, 16)]
                px = jnp.clip(ax_s + dxg, 0, W - 1)
                py = jnp.clip(ay_s + dyg, 0, H - 1)
                nidx_v[kp, pl.ds(g * 16, 16)] = bb_s + py * W + px
            return 0

        lax.fori_loop(0, CK, idx_body, 0)

        sems = (sema, semb, semc, semd, seme, semf, semg, semh)

        def issue(kp):
            for b in range(NBUF):
                @pl.when(kp % NBUF == b)
                def _(b=b):
                    pltpu.async_copy(des2v.at[nidx_v.at[kp]],
                                     rows2_v.at[b], sems[b])

        def drain(kp):
            for b in range(NBUF):
                @pl.when(kp % NBUF == b)
                def _(b=b):
                    pltpu.make_async_copy(des2v.at[pl.ds(0, NP48)],
                                          rows2_v.at[b], sems[b]).wait()

        for k0 in range(NBUF - 1):
            issue(k0)

        def kp_body(kp, carry):
            pos_c, psc_c, ax_c, ay_c, msk_c, q1_c, qof_c = carry
            arow = aux_v[kp, pl.ds(0, 16)]
            ax_s = (_lane(arow, 0) + 0.5).astype(jnp.int32)
            ay_s = (_lane(arow, 1) + 0.5).astype(jnp.int32)
            q1_s = _lane(arow, 2)
            bb_s = _lane(arow, 4).astype(jnp.int32)
            msk_s = (jnp.where(ax_s >= 0, 1, 0) * jnp.where(ay_s >= 0, 1, 0)
                     * jnp.where(ax_s < W, 1, 0) * jnp.where(ay_s < H, 1, 0))

            @pl.when(kp + (NBUF - 1) < CK)
            def _():
                issue(kp + (NBUF - 1))

            drain(kp)
            par = kp % NBUF
            s1c = [s1_v[kp, pl.ds(c * 16, 16)] for c in range(8)]

            def p_body(p, sv):
                v0, v1, v2 = sv
                acc = s1c[0] * rows2_v[par, p, pl.ds(0, 16)]
                for c in range(1, 8):
                    acc = acc + s1c[c] * rows2_v[par, p, pl.ds(c * 16, 16)]
                dot = _bsum(acc)
                v0 = jnp.where(iota == p, dot, v0)
                v1 = jnp.where(iota == p - 16, dot, v1)
                v2 = jnp.where(iota == p - 32, dot, v2)
                return (v0, v1, v2)

            z16 = jnp.zeros((16,), jnp.float32)
            v0, v1, v2 = lax.fori_loop(0, NP48, p_body, (z16, z16, z16))
            nall_v[kp, pl.ds(0, 16)] = v0
            nall_v[kp, pl.ds(16, 16)] = v1
            nall_v[kp, pl.ds(32, 16)] = v2

            neginf = jnp.float32(-3.0e38)
            m1 = jnp.where(iota < (P - 16), v1, neginf)
            psc_s = _bmax(jnp.maximum(v0, m1))
            cand = jnp.minimum(jnp.where(v0 == psc_s, iota, 99),
                               jnp.where(m1 == psc_s, iota + 16, 99))
            pos_s = _bmin(cand)

            dx0 = dxt[pl.ds(0, 16)]
            dx1 = dxt[pl.ds(16, 16)]
            dy0 = dyt[pl.ds(0, 16)]
            dy1 = dyt[pl.ds(16, 16)]
            zi16 = jnp.zeros((16,), jnp.int32)
            dx_s = _bsum(jnp.where(iota == pos_s, dx0, zi16)
                         + jnp.where(iota + 16 == pos_s, dx1, zi16))
            dy_s = _bsum(jnp.where(iota == pos_s, dy0, zi16)
                         + jnp.where(iota + 16 == pos_s, dy1, zi16))
            qof_s = (bb_s + jnp.clip(ay_s + dy_s, 0, H - 1) * W
                     + jnp.clip(ax_s + dx_s, 0, W - 1))

            def upd(c, val):
                return tuple(jnp.where(iota == kp - 16 * g, val, c[g])
                             for g in range(3))

            return (upd(pos_c, pos_s), upd(psc_c, psc_s), upd(ax_c, ax_s),
                    upd(ay_c, ay_s), upd(msk_c, msk_s), upd(q1_c, q1_s),
                    upd(qof_c, qof_s))

        zi = jnp.zeros((16,), jnp.int32)
        zf = jnp.zeros((16,), jnp.float32)
        t3i = (zi, zi, zi)
        t3f = (zf, zf, zf)
        (pos_c, psc_c, ax_c, ay_c, msk_c, q1_c, qof_c) = lax.fori_loop(
            0, CK, kp_body, (t3i, t3f, t3i, t3i, t3i, t3f, t3i))

        for g in range(3):
            sl = pl.ds(g * 16, 16)
            qidx_v[sl] = qof_c[g]
            pscv[sl] = psc_c[g]
            axv[sl] = ax_c[g]
            ayv[sl] = ay_c[g]
            mskv[sl] = msk_c[g]

        pltpu.async_copy(aux.at[qidx_v], q2_v, sem).wait()

        def q_body(kp, qv_c):
            qrow = q2_v[kp, pl.ds(0, 16)]
            q2_s = _lane(qrow, 3)
            zf16 = jnp.zeros((16,), jnp.float32)
            q1_s = _bsum(sum(jnp.where(iota == kp - 16 * g, q1_c[g], zf16)
                             for g in range(3)))
            val = (q1_s + q2_s) * 0.5
            return tuple(jnp.where(iota == kp - 16 * g, val, qv_c[g])
                         for g in range(3))

        qv_c = lax.fori_loop(0, CK, q_body, t3f)
        for g in range(3):
            qltv[pl.ds(g * 16, 16)] = qv_c[g]

        pltpu.sync_copy(nall_v, nallo.at[pl.ds(base, CK)])
        pltpu.sync_copy(pscv, psco.at[pl.ds(base, CK)])
        pltpu.sync_copy(qltv, qlto.at[pl.ds(base, CK)])
        pltpu.sync_copy(mskv, msko.at[pl.ds(base, CK)])
        pltpu.sync_copy(axv, axo.at[pl.ds(base, CK)])
        pltpu.sync_copy(ayv, ayo.at[pl.ds(base, CK)])
        return 0

    lax.fori_loop(0, NCHUNK, chunk_body, 0)


_sc_kernel_cache = None


def _get_sc_kernel():
    global _sc_kernel_cache
    if _sc_kernel_cache is not None:
        return _sc_kernel_cache
    mesh = plsc.VectorSubcoreMesh(core_axis_name="c", subcore_axis_name="s",
                                  num_cores=2, num_subcores=16)
    _sc_kernel_cache = functools.partial(
        pl.kernel,
        compiler_params=pltpu.CompilerParams(use_tc_tiling_on_sc=False),
        out_type=(
        jax.ShapeDtypeStruct((N, NP48), jnp.float32),
        jax.ShapeDtypeStruct((N,), jnp.float32),
        jax.ShapeDtypeStruct((N,), jnp.float32),
        jax.ShapeDtypeStruct((N,), jnp.int32),
        jax.ShapeDtypeStruct((N,), jnp.int32),
        jax.ShapeDtypeStruct((N,), jnp.int32),
        jax.ShapeDtypeStruct((N, D), jnp.float32),
        jax.ShapeDtypeStruct((N, D), jnp.float32),
    ),
        mesh=mesh,
        scratch_types=[
        pltpu.VMEM((CK,), jnp.int32),
        pltpu.VMEM((CK,), jnp.int32),
        pltpu.VMEM((CK, D), jnp.float32),
        pltpu.VMEM((CK, 16), jnp.float32),
        pltpu.VMEM((CK, NP48), jnp.int32),
        pltpu.VMEM((8, NP48, D), jnp.float32),
        pltpu.VMEM((CK, NP48), jnp.float32),
        pltpu.VMEM((CK,), jnp.int32),
        pltpu.VMEM((CK,), jnp.int32),
        pltpu.VMEM((CK,), jnp.int32),
        pltpu.VMEM((CK,), jnp.int32),
        pltpu.VMEM((CK, 16), jnp.float32),
        pltpu.VMEM((CK,), jnp.float32),
        pltpu.VMEM((CK,), jnp.float32),
        pltpu.VMEM((CK,), jnp.int32),
        pltpu.VMEM((NP48,), jnp.int32),
        pltpu.VMEM((NP48,), jnp.int32),
        pltpu.VMEM((CK, D), jnp.float32),
        pltpu.SemaphoreType.DMA,
        pltpu.SemaphoreType.DMA,
        pltpu.SemaphoreType.DMA,
        pltpu.SemaphoreType.DMA,
        pltpu.SemaphoreType.DMA,
        pltpu.SemaphoreType.DMA,
        pltpu.SemaphoreType.DMA,
        pltpu.SemaphoreType.DMA,
        pltpu.SemaphoreType.DMA,
        ],
    )(_sc_body)
    return _sc_kernel_cache


BM = 512


def _score_body(s13_ref, s1_ref, distr_ref, ax_ref, ay_ref, b1_ref,
                xd_ref, yd_ref, bd_ref, out_ref):
    s1 = s1_ref[...]
    dt = distr_ref[...]
    dsc = lax.dot_general(s1, dt, (((1,), (1,)), ((), ())),
                          preferred_element_type=jnp.float32)
    dx = xd_ref[...] - ax_ref[...]
    dy = yd_ref[...] - ay_ref[...]
    dis2 = dx * dx + dy * dy + jnp.where(bd_ref[...] != b1_ref[...], 9, 0)
    dsc = jnp.where(dis2 < 9, jnp.float32(0), dsc)
    out_ref[...] = jnp.concatenate([s13_ref[...][:, :1 + NN], dsc], axis=1)


def _run_scores(s13, s1, distr, ax, ay, b1, xd, yd, bd):
    grid = (N // BM,)
    return pl.pallas_call(
        _score_body,
        grid=grid,
        in_specs=[
            pl.BlockSpec((BM, 16), lambda i: (i, 0)),
            pl.BlockSpec((BM, D), lambda i: (i, 0)),
            pl.BlockSpec((N, D), lambda i: (0, 0)),
            pl.BlockSpec((BM, 1), lambda i: (i, 0)),
            pl.BlockSpec((BM, 1), lambda i: (i, 0)),
            pl.BlockSpec((BM, 1), lambda i: (i, 0)),
            pl.BlockSpec((1, N), lambda i: (0, 0)),
            pl.BlockSpec((1, N), lambda i: (0, 0)),
            pl.BlockSpec((1, N), lambda i: (0, 0)),
        ],
        out_specs=pl.BlockSpec((BM, 1 + NN + N), lambda i: (i, 0)),
        out_shape=jax.ShapeDtypeStruct((N, 1 + NN + N), jnp.float32),
    )(s13, s1, distr, ax, ay, b1, xd, yd, bd)


def kernel(des1, det1, qlt1, des2, det2, qlt2, aflow):
    des1v = des1.transpose(0, 2, 3, 1).reshape(B * HW, D)
    des2v = des2.transpose(0, 2, 3, 1).reshape(B * HW, D)
    bbase = jnp.repeat(jnp.arange(B, dtype=jnp.float32) * HW, HW)
    aux = jnp.concatenate(
        [aflow.transpose(0, 2, 3, 1).reshape(B * HW, 2),
         qlt1.reshape(B * HW, 1), qlt2.reshape(B * HW, 1),
         bbase[:, None], jnp.zeros((B * HW, 11), jnp.float32)], axis=1)
    ptab = jnp.asarray(np.stack([_alldx, _alldy]))

    samp = _run_sample(det1, det2)
    off1 = samp[0, :N]
    off2 = samp[0, N:]
    xd = samp[2, N:][None, :]
    yd = samp[1, N:][None, :]

    nall, psc, qlt, msk, ax, ay, s1, distr = _get_sc_kernel()(
        des1v, des2v, aux, off1, off2, ptab)

    b1 = jnp.repeat(jnp.arange(B, dtype=jnp.int32), NPC)
    s13 = jnp.concatenate(
        [psc[:, None], nall[:, P:P + NN],
         jnp.zeros((N, 3), jnp.float32)], axis=1)
    scores = _run_scores(s13, s1, distr, ax[:, None], ay[:, None],
                         b1[:, None], xd, yd, bd=b1[None, :])

    labels = jnp.zeros(scores.shape, dtype=bool).at[:, :1].set(True)
    mask = msk.astype(bool).reshape(B, NPC)
    return scores, labels, mask, qlt[:, None]
